# Initial kernel scaffold; baseline (speedup 1.0000x reference)
import jax
import jax.numpy as jnp
from jax.experimental import pallas as pl


def kernel(input_ids, token_type_ids, word_embeddings, token_type_embeddings):
    raise NotImplementedError("write your pallas kernel here")


# speedup vs baseline: 2.6376x; 2.6376x over previous
import functools

import jax
import jax.numpy as jnp
from jax import lax
from jax.experimental import pallas as pl
from jax.experimental.pallas import tpu as pltpu
from jax.experimental.pallas import tpu_sc as plsc

NC, NS, L = 2, 16, 16
NW = NC * NS
D = 128
CHUNK = 128


def _emb_body(ids_hbm, tt_hbm, word_hbm, ttab_hbm, out_hbm,
              idx_v, tt_v, rows_v, ttab_v, sem):
    wid = lax.axis_index("s") * NC + lax.axis_index("c")
    n_rows = idx_v.shape[0]
    tpw = n_rows * CHUNK
    row0 = wid * n_rows

    pltpu.sync_copy(ids_hbm.at[pl.ds(row0, n_rows)], idx_v)
    pltpu.sync_copy(tt_hbm.at[pl.ds(wid * tpw, tpw)], tt_v)
    pltpu.sync_copy(ttab_hbm, ttab_v)

    for k in range(n_rows):
        pltpu.async_copy(word_hbm.at[idx_v.at[k]],
                         rows_v.at[pl.ds(k * CHUNK, CHUNK)], sem)
    for k in range(n_rows):
        pltpu.make_async_copy(word_hbm.at[idx_v.at[k]],
                              rows_v.at[pl.ds(k * CHUNK, CHUNK)], sem).wait()

    tt0 = [ttab_v[0, pl.ds(j * L, L)] for j in range(D // L)]
    delta = [ttab_v[1, pl.ds(j * L, L)] - tt0[j] for j in range(D // L)]

    def grp(g, _):
        ttg = tt_v[pl.ds(g * L, L)].astype(jnp.float32)
        for k in range(L):
            i = g * L + k
            sf = ttg[k]
            for j in range(D // L):
                sl = pl.ds(j * L, L)
                rows_v[i, sl] = rows_v[i, sl] + tt0[j] + sf * delta[j]
        return 0

    lax.fori_loop(0, tpw // L, grp, 0)

    pltpu.sync_copy(rows_v, out_hbm.at[pl.ds(wid * tpw, tpw)])


@jax.jit
def _emb(ids2d, ttflat, word, ttab):
    n = ids2d.shape[0] * ids2d.shape[1]
    n_rows_w = ids2d.shape[0] // NW
    mesh = plsc.VectorSubcoreMesh(core_axis_name="c", subcore_axis_name="s",
                                  num_cores=NC, num_subcores=NS)
    f = pl.kernel(
        _emb_body,
        out_type=jax.ShapeDtypeStruct((n, D), jnp.float32),
        mesh=mesh,
        scratch_types=[
            pltpu.VMEM((n_rows_w, CHUNK), jnp.int32),
            pltpu.VMEM((n_rows_w * CHUNK,), jnp.int32),
            pltpu.VMEM((n_rows_w * CHUNK, D), jnp.float32),
            pltpu.VMEM((2, D), jnp.float32),
            pltpu.SemaphoreType.DMA,
        ],
    )
    return f(ids2d, ttflat, word, ttab)


def kernel(input_ids, token_type_ids, word_embeddings, token_type_embeddings):
    b, s = input_ids.shape
    n = b * s
    ids2d = input_ids.reshape(n // CHUNK, CHUNK).astype(jnp.int32)
    ttflat = token_type_ids.reshape(n).astype(jnp.int32)
    out = _emb(ids2d, ttflat, word_embeddings, token_type_embeddings)
    return out.reshape(b, s, D)


# speedup vs baseline: 2.8631x; 1.0855x over previous
import functools

import jax
import jax.numpy as jnp
from jax import lax
from jax.experimental import pallas as pl
from jax.experimental.pallas import tpu as pltpu
from jax.experimental.pallas import tpu_sc as plsc

NC, NS, L = 2, 16, 16
NW = NC * NS
D = 128
CHUNK = 128


def _emb_body(ids_hbm, tt_hbm, word_hbm, ttab_hbm, out_hbm,
              idx_v, tt_v, rows_v, ttab_v, gsem, wsem):
    wid = lax.axis_index("s") * NC + lax.axis_index("c")
    n_rows = idx_v.shape[0]
    tpw = n_rows * CHUNK
    row0 = wid * n_rows

    pltpu.sync_copy(ids_hbm.at[pl.ds(row0, n_rows)], idx_v)
    for k in range(n_rows):
        pltpu.async_copy(word_hbm.at[idx_v.at[k]],
                         rows_v.at[pl.ds(k * CHUNK, CHUNK)], gsem.at[k])

    pltpu.sync_copy(tt_hbm.at[pl.ds(wid * tpw, tpw)], tt_v)
    pltpu.sync_copy(ttab_hbm, ttab_v)

    tt0 = [ttab_v[0, pl.ds(j * L, L)] for j in range(D // L)]
    delta = [ttab_v[1, pl.ds(j * L, L)] - tt0[j] for j in range(D // L)]

    for k in range(n_rows):
        pltpu.make_async_copy(word_hbm.at[idx_v.at[k]],
                              rows_v.at[pl.ds(k * CHUNK, CHUNK)],
                              gsem.at[k]).wait()

        def grp(g, _, k=k):
            ttg = tt_v[pl.ds(k * CHUNK + g * L, L)].astype(jnp.float32)
            for kk in range(L):
                i = k * CHUNK + g * L + kk
                sf = ttg[kk]
                for j in range(D // L):
                    sl = pl.ds(j * L, L)
                    rows_v[i, sl] = rows_v[i, sl] + tt0[j] + sf * delta[j]
            return 0

        lax.fori_loop(0, CHUNK // L, grp, 0)
        pltpu.async_copy(rows_v.at[pl.ds(k * CHUNK, CHUNK)],
                         out_hbm.at[pl.ds(wid * tpw + k * CHUNK, CHUNK)],
                         wsem)

    for k in range(n_rows):
        pltpu.make_async_copy(rows_v.at[pl.ds(k * CHUNK, CHUNK)],
                              out_hbm.at[pl.ds(wid * tpw + k * CHUNK, CHUNK)],
                              wsem).wait()


@jax.jit
def _emb(ids2d, ttflat, word, ttab):
    n = ids2d.shape[0] * ids2d.shape[1]
    n_rows_w = ids2d.shape[0] // NW
    mesh = plsc.VectorSubcoreMesh(core_axis_name="c", subcore_axis_name="s",
                                  num_cores=NC, num_subcores=NS)
    f = pl.kernel(
        _emb_body,
        out_type=jax.ShapeDtypeStruct((n, D), jnp.float32),
        mesh=mesh,
        scratch_types=[
            pltpu.VMEM((n_rows_w, CHUNK), jnp.int32),
            pltpu.VMEM((n_rows_w * CHUNK,), jnp.int32),
            pltpu.VMEM((n_rows_w * CHUNK, D), jnp.float32),
            pltpu.VMEM((2, D), jnp.float32),
            pltpu.SemaphoreType.DMA((n_rows_w,)),
            pltpu.SemaphoreType.DMA,
        ],
    )
    return f(ids2d, ttflat, word, ttab)


def kernel(input_ids, token_type_ids, word_embeddings, token_type_embeddings):
    b, s = input_ids.shape
    n = b * s
    ids2d = input_ids.reshape(n // CHUNK, CHUNK).astype(jnp.int32)
    ttflat = token_type_ids.reshape(n).astype(jnp.int32)
    out = _emb(ids2d, ttflat, word_embeddings, token_type_embeddings)
    return out.reshape(b, s, D)
